# baseline (speedup 1.0000x reference)
import functools

import jax
import jax.numpy as jnp
from jax import lax
from jax.experimental import pallas as pl
from jax.experimental.pallas import tpu as pltpu

B = 32
D = 512
V = 50257

_VB = 8192
_NV = (V + _VB - 1) // _VB


def _mm_body(ids_ref, emb_hbm, wt_ref, b_ref, o_ref, x_vmem, sem):
    v = pl.program_id(0)

    @pl.when(v == 0)
    def _():
        for i in range(B):
            pltpu.make_async_copy(
                emb_hbm.at[pl.ds(ids_ref[i], 1)], x_vmem.at[pl.ds(i, 1)], sem
            ).start()
        for i in range(B):
            pltpu.make_async_copy(
                emb_hbm.at[pl.ds(ids_ref[i], 1)], x_vmem.at[pl.ds(i, 1)], sem
            ).wait()

    res = (
        lax.dot_general(
            x_vmem[...],
            wt_ref[...],
            dimension_numbers=(((1,), (1,)), ((), ())),
            preferred_element_type=jnp.float32,
        )
        + b_ref[...]
    )
    o_ref[...] = res[:, None, :]


def kernel(input_ids, emb, W, b):
    ids = input_ids.reshape(B).astype(jnp.int32)
    grid_spec = pltpu.PrefetchScalarGridSpec(
        num_scalar_prefetch=1,
        grid=(_NV,),
        in_specs=[
            pl.BlockSpec(memory_space=pltpu.MemorySpace.HBM),
            pl.BlockSpec((_VB, D), lambda v, ids: (v, 0)),
            pl.BlockSpec((1, _VB), lambda v, ids: (0, v)),
        ],
        out_specs=pl.BlockSpec((B, 1, _VB), lambda v, ids: (0, 0, v)),
        scratch_shapes=[
            pltpu.VMEM((B, D), jnp.float32),
            pltpu.SemaphoreType.DMA,
        ],
    )
    out = pl.pallas_call(
        _mm_body,
        grid_spec=grid_spec,
        out_shape=jax.ShapeDtypeStruct((B, 1, V), jnp.float32),
        compiler_params=pltpu.CompilerParams(
            dimension_semantics=("arbitrary",),
        ),
    )(ids, emb, W.T, b.reshape(1, V))
    return out


# speedup vs baseline: 1.0936x; 1.0936x over previous
import functools

import jax
import jax.numpy as jnp
from jax import lax
from jax.experimental import pallas as pl
from jax.experimental.pallas import tpu as pltpu

B = 32
D = 512
V = 50257

_VB = 4096
_NV = (V + _VB - 1) // _VB


def _mm_body(ids_ref, emb_hbm, wt_ref, b_ref, o_ref, x_vmem, sem):
    v = pl.program_id(0)

    @pl.when(v == 0)
    def _():
        for i in range(B):
            pltpu.make_async_copy(
                emb_hbm.at[pl.ds(ids_ref[i], 1)], x_vmem.at[pl.ds(i, 1)], sem
            ).start()
        for i in range(B):
            pltpu.make_async_copy(
                emb_hbm.at[pl.ds(ids_ref[i], 1)], x_vmem.at[pl.ds(i, 1)], sem
            ).wait()

    res = (
        lax.dot_general(
            x_vmem[...],
            wt_ref[...],
            dimension_numbers=(((1,), (1,)), ((), ())),
            preferred_element_type=jnp.float32,
        )
        + b_ref[...]
    )
    o_ref[...] = res[:, None, :]


def kernel(input_ids, emb, W, b):
    ids = input_ids.reshape(B).astype(jnp.int32)
    grid_spec = pltpu.PrefetchScalarGridSpec(
        num_scalar_prefetch=1,
        grid=(_NV,),
        in_specs=[
            pl.BlockSpec(memory_space=pltpu.MemorySpace.HBM),
            pl.BlockSpec((_VB, D), lambda v, ids: (v, 0)),
            pl.BlockSpec((1, _VB), lambda v, ids: (0, v)),
        ],
        out_specs=pl.BlockSpec((B, 1, _VB), lambda v, ids: (0, 0, v)),
        scratch_shapes=[
            pltpu.VMEM((B, D), jnp.float32),
            pltpu.SemaphoreType.DMA,
        ],
    )
    out = pl.pallas_call(
        _mm_body,
        grid_spec=grid_spec,
        out_shape=jax.ShapeDtypeStruct((B, 1, V), jnp.float32),
        compiler_params=pltpu.CompilerParams(
            dimension_semantics=("arbitrary",),
        ),
    )(ids, emb, W.T, b.reshape(1, V))
    return out
